# baseline (speedup 1.0000x reference)
import numpy as np
import jax
import jax.numpy as jnp
from jax import lax
from jax.experimental import pallas as pl
from jax.experimental.pallas import tpu as pltpu
from jax.experimental.pallas import tpu_sc as plsc

VOCAB_N = 1000000
D = 64
BATCH = 4096
SEQ = 200
MAXLEN = 4096

NW = 32
TOTAL = BATCH * SEQ
RPW = TOTAL // NW
C = SEQ
NCHUNK = RPW // C
NB = 6
LOOKAHEAD = 3
PE_ROWS = SEQ


def _make_pe_rep():
    position = np.arange(MAXLEN, dtype=np.float32)[:, None]
    div_term = np.exp(
        np.arange(0, D, 2, dtype=np.float32) * (-np.log(10000.0) / D))
    pe = np.zeros((MAXLEN, D), dtype=np.float32)
    pe[:, 0::2] = np.sin(position * div_term)
    pe[:, 1::2] = np.cos(position * div_term)
    pe = pe[:SEQ]
    return pe


_PE_REP = jnp.asarray(_make_pe_rep())


def _sc_embed(x3, table, pe_rep):
    mesh = plsc.VectorSubcoreMesh(core_axis_name="c", subcore_axis_name="s")

    @pl.kernel(
        out_type=jax.ShapeDtypeStruct((BATCH, SEQ, D), jnp.float32),
        mesh=mesh,
        compiler_params=pltpu.CompilerParams(use_tc_tiling_on_sc=False),
        scratch_types=[
            pltpu.VMEM((NCHUNK, C), jnp.int32),
            pltpu.VMEM((PE_ROWS, D), jnp.float32),
            pltpu.VMEM((NB, C, D), jnp.float32),
            pltpu.SemaphoreType.DMA((NB,)),
            pltpu.SemaphoreType.DMA((NB,)),
        ],
    )
    def k(x_hbm, pe_hbm, table_hbm, out_hbm,
          idx_v, pe_v, rows_v, gsem, ssem):
        sid = lax.axis_index("s")
        wid = sid * 2 + lax.axis_index("c")
        pltpu.sync_copy(x_hbm.at[wid], idx_v)
        pltpu.sync_copy(pe_hbm, pe_v)

        def issue_gather(c):
            b = c % NB
            return pltpu.async_copy(table_hbm.at[idx_v.at[c]],
                                    rows_v.at[b], gsem.at[b])

        def issue_store(c):
            b = c % NB
            seq_i = wid * NCHUNK + c
            return pltpu.async_copy(rows_v.at[b],
                                    out_hbm.at[seq_i], ssem.at[b])

        def add_pe(c):
            b = c % NB
            rv = rows_v.at[b]

            @pl.loop(0, C)
            def _(r):
                for kk in range(D // 16):
                    sl = pl.ds(16 * kk, 16)
                    rv[pl.ds(r, 1), sl] = (rv[pl.ds(r, 1), sl]
                                           + pe_v[pl.ds(r, 1), sl])

        gathers, stores = {}, {}
        for t in range(NCHUNK + LOOKAHEAD):
            if t < NCHUNK:
                prev = t - NB
                if prev >= 0:
                    stores.pop(prev).wait()
                gathers[t] = issue_gather(t)
            c = t - LOOKAHEAD
            if c >= 0:
                gathers.pop(c).wait()
                add_pe(c)
                stores[c] = issue_store(c)

        for h in stores.values():
            h.wait()

    return k(x3, pe_rep, table)


def kernel(x, table):
    x3 = x.astype(jnp.int32).reshape(NW, NCHUNK, C)
    return _sc_embed(x3, table, _PE_REP)
